# baseline (speedup 1.0000x reference)
import functools

import jax
import jax.numpy as jnp
from jax import lax
from jax.experimental import pallas as pl
from jax.experimental.pallas import tpu as pltpu
from jax.experimental.pallas import tpu_sc as plsc

E = 160000
D = 128
S = 5
R = 2 * S + 2
C = 40
G = 4
GROWS = R * C // G
NCH = E // C
NW = 32
CPW = NCH // NW


def _make_sc_kernel():
    mesh = plsc.VectorSubcoreMesh(core_axis_name="c", subcore_axis_name="s")

    @functools.partial(
        pl.kernel,
        mesh=mesh,
        out_type=(
            jax.ShapeDtypeStruct((E, 2 * D), jnp.float32),
            jax.ShapeDtypeStruct((E, 2 * D), jnp.float32),
        ),
        scratch_types=[
            pltpu.VMEM((G, GROWS), jnp.int32),
            pltpu.VMEM((R * C, D), jnp.float32),
            pltpu.VMEM((C, D), jnp.float32),
            pltpu.VMEM((C, D), jnp.float32),
            pltpu.SemaphoreType.DMA,
            pltpu.SemaphoreType.DMA,
        ],
    )
    def k(x_hbm, idx_hbm, ea_hbm, nb_hbm, idx_v, buf, nbl, nbr, gsem, osem):
        wid = lax.axis_index("s") * 2 + lax.axis_index("c")

        def chunk_body(j, carry):
            chunk = wid * CPW + j
            base = chunk * C
            pltpu.sync_copy(idx_hbm.at[chunk], idx_v)
            gathers = [
                pltpu.async_copy(
                    x_hbm.at[idx_v.at[g]],
                    buf.at[pl.ds(g * GROWS, GROWS)],
                    gsem,
                )
                for g in range(G)
            ]
            for cp in gathers:
                cp.wait()
            out1 = pltpu.async_copy(
                buf.at[pl.ds(0, C)],
                ea_hbm.at[pl.ds(base, C), pl.ds(0, D)],
                osem,
            )
            out2 = pltpu.async_copy(
                buf.at[pl.ds(C, C)],
                ea_hbm.at[pl.ds(base, C), pl.ds(D, D)],
                osem,
            )

            def cbody(c, cc):
                for v in range(D // 16):
                    sl = pl.ds(v * 16, 16)
                    s_ = buf[c, sl]
                    d_ = buf[C + c, sl]
                    nbl[c, sl] = (s_ + d_) * 0.5
                    acc = buf[2 * C + c * S, sl]
                    for r in range(1, S):
                        acc = acc + buf[2 * C + c * S + r, sl]
                    for r in range(S):
                        acc = acc + buf[2 * C + C * S + c * S + r, sl]
                    nbr[c, sl] = acc * jnp.float32(1.0 / (2 * S))
                return cc

            lax.fori_loop(0, C, cbody, 0)

            out3 = pltpu.async_copy(
                nbl, nb_hbm.at[pl.ds(base, C), pl.ds(0, D)], osem)
            out4 = pltpu.async_copy(
                nbr, nb_hbm.at[pl.ds(base, C), pl.ds(D, D)], osem)
            out1.wait()
            out2.wait()
            out3.wait()
            out4.wait()
            return carry

        lax.fori_loop(0, CPW, chunk_body, 0)

    return k


_sc_agg = _make_sc_kernel()


def kernel(x, edge_index, nb_idx):
    src = edge_index[0]
    dst = edge_index[1]
    idx_ch = jnp.concatenate(
        [src.reshape(NCH, C), dst.reshape(NCH, C),
         nb_idx[0].reshape(NCH, C * S), nb_idx[1].reshape(NCH, C * S)],
        axis=1,
    ).reshape(NCH, G, GROWS)
    ea, nb = _sc_agg(x, idx_ch)
    return ea, nb


# speedup vs baseline: 1.5059x; 1.5059x over previous
import functools

import jax
import jax.numpy as jnp
from jax import lax
from jax.experimental import pallas as pl
from jax.experimental.pallas import tpu as pltpu
from jax.experimental.pallas import tpu_sc as plsc

E = 160000
D = 128
S = 5
R = 2 * S + 2
C = 40
G = 4
GROWS = R * C // G
NCH = E // C
NW = 32
CPW = NCH // NW


def _make_sc_kernel():
    mesh = plsc.VectorSubcoreMesh(core_axis_name="c", subcore_axis_name="s")

    @functools.partial(
        pl.kernel,
        mesh=mesh,
        out_type=(
            jax.ShapeDtypeStruct((E, 2 * D), jnp.float32),
            jax.ShapeDtypeStruct((E, 2 * D), jnp.float32),
        ),
        scratch_types=[
            pltpu.VMEM((G, GROWS), jnp.int32),
            pltpu.VMEM((R * C, D), jnp.float32),
            pltpu.VMEM((C, D), jnp.float32),
            pltpu.VMEM((C, D), jnp.float32),
            pltpu.SemaphoreType.DMA,
            pltpu.SemaphoreType.DMA,
        ],
    )
    def k(x_hbm, idx_hbm, ea_hbm, nb_hbm, idx_v, buf, nbl, nbr, gsem, osem):
        wid = lax.axis_index("s") * 2 + lax.axis_index("c")

        def chunk_body(j, carry):
            chunk = wid * CPW + j
            base = chunk * C
            pltpu.sync_copy(idx_hbm.at[chunk], idx_v)
            gathers = [
                pltpu.async_copy(
                    x_hbm.at[idx_v.at[g]],
                    buf.at[pl.ds(g * GROWS, GROWS)],
                    gsem,
                )
                for g in range(G)
            ]
            for cp in gathers:
                cp.wait()
            out1 = pltpu.async_copy(
                buf.at[pl.ds(0, C)],
                ea_hbm.at[pl.ds(base, C), pl.ds(0, D)],
                osem,
            )
            out2 = pltpu.async_copy(
                buf.at[pl.ds(C, C)],
                ea_hbm.at[pl.ds(base, C), pl.ds(D, D)],
                osem,
            )

            def cbody(c, cc):
                for v in range(D // 16):
                    sl = pl.ds(v * 16, 16)
                    s_ = buf[c, sl]
                    d_ = buf[C + c, sl]
                    nbl[c, sl] = (s_ + d_) * 0.5
                    acc = buf[2 * C + c, sl]
                    for r in range(3, R):
                        acc = acc + buf[r * C + c, sl]
                    nbr[c, sl] = acc * jnp.float32(1.0 / (2 * S))
                return cc

            lax.fori_loop(0, C, cbody, 0)

            out3 = pltpu.async_copy(
                nbl, nb_hbm.at[pl.ds(base, C), pl.ds(0, D)], osem)
            out4 = pltpu.async_copy(
                nbr, nb_hbm.at[pl.ds(base, C), pl.ds(D, D)], osem)
            out1.wait()
            out2.wait()
            out3.wait()
            out4.wait()
            return carry

        lax.fori_loop(0, CPW, chunk_body, 0)

    return k


_sc_agg = _make_sc_kernel()


def kernel(x, edge_index, nb_idx):
    src = edge_index[0]
    dst = edge_index[1]
    idx_ch = jnp.concatenate(
        [src.reshape(NCH, 1, C), dst.reshape(NCH, 1, C),
         nb_idx[0].reshape(NCH, C, S).transpose(0, 2, 1),
         nb_idx[1].reshape(NCH, C, S).transpose(0, 2, 1)],
        axis=1,
    ).reshape(NCH, G, GROWS)
    ea, nb = _sc_agg(x, idx_ch)
    return ea, nb


# speedup vs baseline: 1.6980x; 1.1276x over previous
import functools

import jax
import jax.numpy as jnp
from jax import lax
from jax.experimental import pallas as pl
from jax.experimental.pallas import tpu as pltpu
from jax.experimental.pallas import tpu_sc as plsc

E = 160000
D = 128
S = 5
R = 2 * S + 2
C = 40
G = 4
GROWS = R * C // G
NCH = E // C
NW = 32
CPW = NCH // NW
K = 25
NG = CPW // K


def _make_sc_kernel():
    mesh = plsc.VectorSubcoreMesh(core_axis_name="c", subcore_axis_name="s")

    @functools.partial(
        pl.kernel,
        mesh=mesh,
        out_type=(
            jax.ShapeDtypeStruct((E, 2 * D), jnp.float32),
            jax.ShapeDtypeStruct((E, 2 * D), jnp.float32),
        ),
        scratch_types=[
            pltpu.VMEM((2, K, G, GROWS), jnp.int32),
            pltpu.VMEM((R * C, D), jnp.float32),
            pltpu.VMEM((C, D), jnp.float32),
            pltpu.VMEM((C, D), jnp.float32),
            pltpu.SemaphoreType.DMA,
            pltpu.SemaphoreType.DMA,
            pltpu.SemaphoreType.DMA,
            pltpu.SemaphoreType.DMA,
        ],
    )
    def k(x_hbm, idx_hbm, ea_hbm, nb_hbm,
          idx2, buf, nbl, nbr, isem0, isem1, gsem, osem):
        wid = lax.axis_index("s") * 2 + lax.axis_index("c")
        base0 = wid * CPW
        isems = [isem0, isem1]

        pending = pltpu.async_copy(
            idx_hbm.at[pl.ds(base0, K)], idx2.at[0], isems[0])

        for g in range(NG):
            slot = g % 2
            nxt = None
            if g + 1 < NG:
                nxt = pltpu.async_copy(
                    idx_hbm.at[pl.ds(base0 + (g + 1) * K, K)],
                    idx2.at[1 - slot], isems[1 - slot])
            pending.wait()
            pending = nxt

            def chunk_body(jj, carry, slot=slot, g=g):
                base = (base0 + g * K + jj) * C
                gathers = [
                    pltpu.async_copy(
                        x_hbm.at[idx2.at[slot, jj, gt]],
                        buf.at[pl.ds(gt * GROWS, GROWS)],
                        gsem,
                    )
                    for gt in range(G)
                ]
                for cp in gathers:
                    cp.wait()
                out1 = pltpu.async_copy(
                    buf.at[pl.ds(0, C)],
                    ea_hbm.at[pl.ds(base, C), pl.ds(0, D)],
                    osem,
                )
                out2 = pltpu.async_copy(
                    buf.at[pl.ds(C, C)],
                    ea_hbm.at[pl.ds(base, C), pl.ds(D, D)],
                    osem,
                )

                def cbody(c, cc):
                    for v in range(D // 16):
                        sl = pl.ds(v * 16, 16)
                        s_ = buf[c, sl]
                        d_ = buf[C + c, sl]
                        nbl[c, sl] = (s_ + d_) * 0.5
                        acc = buf[2 * C + c, sl]
                        for r in range(3, R):
                            acc = acc + buf[r * C + c, sl]
                        nbr[c, sl] = acc * jnp.float32(1.0 / (2 * S))
                    return cc

                lax.fori_loop(0, C, cbody, 0)

                out3 = pltpu.async_copy(
                    nbl, nb_hbm.at[pl.ds(base, C), pl.ds(0, D)], osem)
                out4 = pltpu.async_copy(
                    nbr, nb_hbm.at[pl.ds(base, C), pl.ds(D, D)], osem)
                out1.wait()
                out2.wait()
                out3.wait()
                out4.wait()
                return carry

            lax.fori_loop(0, K, chunk_body, 0)

    return k


_sc_agg = _make_sc_kernel()


def kernel(x, edge_index, nb_idx):
    src = edge_index[0]
    dst = edge_index[1]
    idx_full = jnp.concatenate(
        [src[None, :], dst[None, :],
         jnp.transpose(nb_idx[0]), jnp.transpose(nb_idx[1])],
        axis=0,
    )
    idx_ch = (
        idx_full.reshape(R, NCH, C)
        .transpose(1, 0, 2)
        .reshape(NCH, G, GROWS)
    )
    ea, nb = _sc_agg(x, idx_ch)
    return ea, nb


# speedup vs baseline: 1.7295x; 1.0185x over previous
import functools

import jax
import jax.numpy as jnp
from jax import lax
from jax.experimental import pallas as pl
from jax.experimental.pallas import tpu as pltpu
from jax.experimental.pallas import tpu_sc as plsc

E = 160000
D = 128
S = 5
R = 2 * S + 2
C = 40
G = 4
GROWS = R * C // G
NCH = E // C
NW = 32
CPW = NCH // NW
K = 25
NG = CPW // K


def _make_sc_kernel():
    mesh = plsc.VectorSubcoreMesh(core_axis_name="c", subcore_axis_name="s")

    @functools.partial(
        pl.kernel,
        mesh=mesh,
        out_type=(
            jax.ShapeDtypeStruct((E, 2 * D), jnp.float32),
            jax.ShapeDtypeStruct((E, 2 * D), jnp.float32),
        ),
        scratch_types=[
            pltpu.VMEM((2, K, G, GROWS), jnp.int32),
            pltpu.VMEM((R * C, D), jnp.float32),
            pltpu.VMEM((2, C, D), jnp.float32),
            pltpu.VMEM((2, C, D), jnp.float32),
            pltpu.SemaphoreType.DMA,
            pltpu.SemaphoreType.DMA,
            pltpu.SemaphoreType.DMA,
            pltpu.SemaphoreType.DMA,
            pltpu.SemaphoreType.DMA,
        ],
    )
    def k(x_hbm, idx_hbm, ea_hbm, nb_hbm,
          idx2, buf, nbl, nbr, isem0, isem1, gsem, oseme, osemn):
        wid = lax.axis_index("s") * 2 + lax.axis_index("c")
        base0 = wid * CPW
        isems = [isem0, isem1]

        pending = pltpu.async_copy(
            idx_hbm.at[pl.ds(base0, K)], idx2.at[0], isems[0])

        for g in range(NG):
            slot = g % 2
            nxt = None
            if g + 1 < NG:
                nxt = pltpu.async_copy(
                    idx_hbm.at[pl.ds(base0 + (g + 1) * K, K)],
                    idx2.at[1 - slot], isems[1 - slot])
            pending.wait()
            pending = nxt
            gb = base0 + g * K

            def emit_chunk(jj, si, slot=slot, gb=gb):
                base = (gb + jj) * C
                gathers = [
                    pltpu.async_copy(
                        x_hbm.at[idx2.at[slot, jj, gt]],
                        buf.at[pl.ds(gt * GROWS, GROWS)],
                        gsem,
                    )
                    for gt in range(G)
                ]
                for cp in gathers:
                    cp.wait()
                o1 = pltpu.async_copy(
                    buf.at[pl.ds(0, C)],
                    ea_hbm.at[pl.ds(base, C), pl.ds(0, D)],
                    oseme,
                )
                o2 = pltpu.async_copy(
                    buf.at[pl.ds(C, C)],
                    ea_hbm.at[pl.ds(base, C), pl.ds(D, D)],
                    oseme,
                )

                def cbody(c, cc):
                    for v in range(D // 16):
                        sl = pl.ds(v * 16, 16)
                        s_ = buf[c, sl]
                        d_ = buf[C + c, sl]
                        nbl[si, c, sl] = (s_ + d_) * 0.5
                        acc = buf[2 * C + c, sl]
                        for r in range(3, R):
                            acc = acc + buf[r * C + c, sl]
                        nbr[si, c, sl] = acc * jnp.float32(1.0 / (2 * S))
                    return cc

                lax.fori_loop(0, C, cbody, 0)

                o3 = pltpu.async_copy(
                    nbl.at[si], nb_hbm.at[pl.ds(base, C), pl.ds(0, D)], osemn)
                o4 = pltpu.async_copy(
                    nbr.at[si], nb_hbm.at[pl.ds(base, C), pl.ds(D, D)], osemn)
                return o1, o2, o3, o4

            def pair_body(p, carry):
                jj = 2 * p
                o1a, o2a, o3a, o4a = emit_chunk(jj, 0)
                o1a.wait()
                o2a.wait()
                o1b, o2b, o3b, o4b = emit_chunk(jj + 1, 1)
                o3a.wait()
                o4a.wait()
                o1b.wait()
                o2b.wait()
                o3b.wait()
                o4b.wait()
                return carry

            lax.fori_loop(0, K // 2, pair_body, 0)
            o1, o2, o3, o4 = emit_chunk(K - 1, 0)
            o1.wait()
            o2.wait()
            o3.wait()
            o4.wait()

    return k


_sc_agg = _make_sc_kernel()


def kernel(x, edge_index, nb_idx):
    src = edge_index[0]
    dst = edge_index[1]
    idx_full = jnp.concatenate(
        [src[None, :], dst[None, :],
         jnp.transpose(nb_idx[0]), jnp.transpose(nb_idx[1])],
        axis=0,
    )
    idx_ch = (
        idx_full.reshape(R, NCH, C)
        .transpose(1, 0, 2)
        .reshape(NCH, G, GROWS)
    )
    ea, nb = _sc_agg(x, idx_ch)
    return ea, nb


# speedup vs baseline: 1.7323x; 1.0016x over previous
import functools

import jax
import jax.numpy as jnp
from jax import lax
from jax.experimental import pallas as pl
from jax.experimental.pallas import tpu as pltpu
from jax.experimental.pallas import tpu_sc as plsc

E = 160000
D = 128
S = 5
R = 2 * S + 2
C = 40
G = 4
GROWS = R * C // G
NCH = E // C
NW = 32
CPW = NCH // NW
K = 25
NG = CPW // K


def _make_sc_kernel():
    mesh = plsc.VectorSubcoreMesh(core_axis_name="c", subcore_axis_name="s")

    @functools.partial(
        pl.kernel,
        mesh=mesh,
        out_type=(
            jax.ShapeDtypeStruct((E, 2 * D), jnp.float32),
            jax.ShapeDtypeStruct((E, 2 * D), jnp.float32),
        ),
        scratch_types=[
            pltpu.VMEM((2, K, G, GROWS), jnp.int32),
            pltpu.VMEM((R * C, D), jnp.float32),
            pltpu.VMEM((2, C, D), jnp.float32),
            pltpu.VMEM((2, C, D), jnp.float32),
            pltpu.SemaphoreType.DMA,
            pltpu.SemaphoreType.DMA,
            pltpu.SemaphoreType.DMA,
            pltpu.SemaphoreType.DMA,
            pltpu.SemaphoreType.DMA,
        ],
    )
    def k(x_hbm, idx_hbm, ea_hbm, nb_hbm,
          idx2, buf, nbl, nbr, isem0, isem1, gsem, oseme, osemn):
        wid = lax.axis_index("s") * 2 + lax.axis_index("c")
        base0 = wid * CPW
        isems = [isem0, isem1]

        pending = pltpu.async_copy(
            idx_hbm.at[pl.ds(base0, K)], idx2.at[0], isems[0])

        for g in range(NG):
            slot = g % 2
            nxt = None
            if g + 1 < NG:
                nxt = pltpu.async_copy(
                    idx_hbm.at[pl.ds(base0 + (g + 1) * K, K)],
                    idx2.at[1 - slot], isems[1 - slot])
            pending.wait()
            pending = nxt
            gb = base0 + g * K

            def fire_gather(jj, gt, slot=slot):
                return pltpu.async_copy(
                    x_hbm.at[idx2.at[slot, jj, gt]],
                    buf.at[pl.ds(gt * GROWS, GROWS)],
                    gsem,
                )

            def process(jj, si, gathers, gb=gb):
                base = (gb + jj) * C
                for cp in gathers:
                    cp.wait()
                o1 = pltpu.async_copy(
                    buf.at[pl.ds(0, C)],
                    ea_hbm.at[pl.ds(base, C), pl.ds(0, D)],
                    oseme,
                )
                o2 = pltpu.async_copy(
                    buf.at[pl.ds(C, C)],
                    ea_hbm.at[pl.ds(base, C), pl.ds(D, D)],
                    oseme,
                )

                def cbody(c, cc):
                    for v in range(D // 16):
                        sl = pl.ds(v * 16, 16)
                        s_ = buf[c, sl]
                        d_ = buf[C + c, sl]
                        nbl[si, c, sl] = (s_ + d_) * 0.5
                        acc = buf[2 * C + c, sl]
                        for r in range(3, R):
                            acc = acc + buf[r * C + c, sl]
                        nbr[si, c, sl] = acc * jnp.float32(1.0 / (2 * S))
                    return cc

                lax.fori_loop(0, C, cbody, 0)

                o3 = pltpu.async_copy(
                    nbl.at[si], nb_hbm.at[pl.ds(base, C), pl.ds(0, D)], osemn)
                o4 = pltpu.async_copy(
                    nbr.at[si], nb_hbm.at[pl.ds(base, C), pl.ds(D, D)], osemn)
                return o1, o2, o3, o4

            def pair_body(p, carry):
                jj = 2 * p
                ga = [fire_gather(jj, gt) for gt in range(G)]
                o1a, o2a, o3a, o4a = process(jj, 0, ga)
                gb_late = [fire_gather(jj + 1, gt) for gt in range(1, G)]
                o1a.wait()
                o2a.wait()
                gb0 = fire_gather(jj + 1, 0)
                o1b, o2b, o3b, o4b = process(jj + 1, 1, [gb0] + gb_late)
                o3a.wait()
                o4a.wait()
                o1b.wait()
                o2b.wait()
                o3b.wait()
                o4b.wait()
                return carry

            lax.fori_loop(0, K // 2, pair_body, 0)
            o1, o2, o3, o4 = process(
                K - 1, 0, [fire_gather(K - 1, gt) for gt in range(G)])
            o1.wait()
            o2.wait()
            o3.wait()
            o4.wait()

    return k


_sc_agg = _make_sc_kernel()


def kernel(x, edge_index, nb_idx):
    src = edge_index[0]
    dst = edge_index[1]
    idx_full = jnp.concatenate(
        [src[None, :], dst[None, :],
         jnp.transpose(nb_idx[0]), jnp.transpose(nb_idx[1])],
        axis=0,
    )
    idx_ch = (
        idx_full.reshape(R, NCH, C)
        .transpose(1, 0, 2)
        .reshape(NCH, G, GROWS)
    )
    ea, nb = _sc_agg(x, idx_ch)
    return ea, nb
